# Initial kernel scaffold; baseline (speedup 1.0000x reference)
import jax
import jax.numpy as jnp
from jax.experimental import pallas as pl


def kernel(x, edge_index, conv_w0, conv_b0, conv_w1, conv_b1, lin_w, lin_b):
    raise NotImplementedError("write your pallas kernel here")


# speedup vs baseline: 16.1670x; 16.1670x over previous
import jax
import jax.numpy as jnp
from jax import lax
from jax.experimental import pallas as pl
from jax.experimental.pallas import tpu as pltpu
from jax.experimental.pallas import tpu_sc as plsc

N = 10000
D = 128
E = 320000
NC = 2
NS = 16
NW = NC * NS
EPW = E // NW
K = 40
NCHUNK = EPW // K
S = 10
NSUP = NCHUNK // S
NB = 5
NGRP = S // NB
HS = 2000
NHS = EPW // HS
SLAB_STEP = 624
SLAB = 640


def _make_seg(with_counts):
    def body(*refs):
        if with_counts:
            (h_hbm, src_hbm, dst_hbm, dstf_hbm, p_hbm, cnt_hbm, accum,
             msg0, msg1, msg2, msg3, msg4, sidx0, sidx1, didx0, didx1,
             hist, hb, gsem, ssem, isem) = refs
        else:
            (h_hbm, src_hbm, dst_hbm, p_hbm, accum,
             msg0, msg1, msg2, msg3, msg4, sidx0, sidx1, didx0, didx1,
             gsem, ssem, isem) = refs
        c = lax.axis_index("c")
        s = lax.axis_index("s")
        wid = s * NC + c
        row0 = s * SLAB_STEP
        msg = (msg0, msg1, msg2, msg3, msg4)
        sidx = (sidx0, sidx1)
        didx = (didx0, didx1)

        def pref_idx(sup):
            b = sup % 2
            pltpu.async_copy(src_hbm.at[wid, sup], sidx[b], isem)
            pltpu.async_copy(dst_hbm.at[wid, sup], didx[b], isem)

        def wait_idx(sup):
            b = sup % 2
            pltpu.make_async_copy(src_hbm.at[wid, sup], sidx[b], isem).wait()
            pltpu.make_async_copy(dst_hbm.at[wid, sup], didx[b], isem).wait()

        def gather(sb, g, b):
            return pltpu.async_copy(h_hbm.at[sb.at[g]], msg[b], gsem)

        def scat(db, g, b):
            pltpu.async_copy(msg[b], accum.at[db.at[g]], ssem, add=True)

        def wait_scat():
            pltpu.make_async_copy(msg[0], accum.at[didx0.at[0]], ssem).wait()

        pref_idx(0)
        pltpu.sync_copy(h_hbm.at[pl.ds(row0, SLAB)],
                        accum.at[pl.ds(row0, SLAB)])
        plsc.subcore_barrier()

        for sup in range(NSUP):
            sb, db = sidx[sup % 2], didx[sup % 2]
            wait_idx(sup)
            cps = []
            for b in range(NB):
                if sup > 0:
                    wait_scat()
                cps.append(gather(sb, b, b))
            for b in range(NB):
                cps[b].wait()
                scat(db, b, b)
            if sup + 1 < NSUP:
                pref_idx(sup + 1)

            def grp(i, _, sb=sb, db=db):
                g0 = NB * i
                cps = []
                for b in range(NB):
                    wait_scat()
                    cps.append(gather(sb, g0 + b, b))
                for b in range(NB):
                    cps[b].wait()
                    scat(db, g0 + b, b)
                return 0

            lax.fori_loop(1, NGRP, grp, 0)

        for b in range(NB):
            wait_scat()

        if with_counts:
            def zero(i, _):
                hist[0, pl.ds(i * 16, 16)] = jnp.zeros((16,), jnp.float32)
                return 0
            lax.fori_loop(0, N // 16, zero, 0)
            ones16 = jnp.ones((16,), jnp.float32)
            zero16 = jnp.zeros((16,), jnp.int32)
            for j in range(NHS):
                pltpu.sync_copy(dstf_hbm.at[wid, j], hb)

                def acc(i, _):
                    plsc.addupdate_scatter(
                        hist, [zero16, hb[0, pl.ds(i * 16, 16)]], ones16)
                    return 0
                lax.fori_loop(0, HS // 16, acc, 0)
            pltpu.sync_copy(hist, cnt_hbm.at[wid])

        plsc.subcore_barrier()
        pltpu.sync_copy(accum.at[pl.ds(row0, SLAB)],
                        p_hbm.at[c].at[pl.ds(row0, SLAB)])

    out_type = [jax.ShapeDtypeStruct((NC, N, D), jnp.float32)]
    scratch = [
        pltpu.VMEM_SHARED((N, D), jnp.float32),
        pltpu.VMEM((K, D), jnp.float32),
        pltpu.VMEM((K, D), jnp.float32),
        pltpu.VMEM((K, D), jnp.float32),
        pltpu.VMEM((K, D), jnp.float32),
        pltpu.VMEM((K, D), jnp.float32),
        pltpu.VMEM((S, K), jnp.int32),
        pltpu.VMEM((S, K), jnp.int32),
        pltpu.VMEM((S, K), jnp.int32),
        pltpu.VMEM((S, K), jnp.int32),
    ]
    if with_counts:
        out_type = out_type + [jax.ShapeDtypeStruct((NW, 1, N), jnp.float32)]
        scratch = scratch + [
            pltpu.VMEM((1, N), jnp.float32),
            pltpu.VMEM((1, HS), jnp.int32),
        ]
    scratch = scratch + [
        pltpu.SemaphoreType.DMA,
        pltpu.SemaphoreType.DMA,
        pltpu.SemaphoreType.DMA,
    ]
    return pl.kernel(
        body,
        out_type=tuple(out_type) if with_counts else out_type[0],
        mesh=plsc.VectorSubcoreMesh(core_axis_name="c", subcore_axis_name="s"),
        compiler_params=pltpu.CompilerParams(needs_layout_passes=False),
        scratch_types=scratch,
    )


_seg_counts = _make_seg(True)
_seg_plain = _make_seg(False)


def _cnt_recip(cnt_ref):
    ones = jnp.ones((NW, 8), jnp.float32)
    cnt = lax.dot_general(cnt_ref[...], ones, (((0,), (0,)), ((), ())),
                          preferred_element_type=jnp.float32)
    return 1.0 / (cnt[:, 0:1] + 1.0)


def _conv_body(x_ref, c_ref, b_ref, o_ref):
    o_ref[...] = (jnp.dot(x_ref[...], c_ref[...],
                          preferred_element_type=jnp.float32) + b_ref[...])


_conv = pl.pallas_call(
    _conv_body,
    out_shape=jax.ShapeDtypeStruct((N, D), jnp.float32),
)


def _comb1_body(p_ref, cnt_ref, h_ref, c_ref, b_ref, o_ref):
    rec = _cnt_recip(cnt_ref)
    m = jnp.maximum((p_ref[0] + p_ref[1] - h_ref[...]) * rec, 0.0)
    o_ref[...] = (jnp.dot(m, c_ref[...],
                          preferred_element_type=jnp.float32) + b_ref[...])


_comb1 = pl.pallas_call(
    _comb1_body,
    out_shape=jax.ShapeDtypeStruct((N, D), jnp.float32),
)


def _comb2_body(p_ref, cnt_ref, h_ref, o_ref):
    rec = _cnt_recip(cnt_ref)
    o_ref[...] = jnp.maximum((p_ref[0] + p_ref[1] - h_ref[...]) * rec, 0.0)


_comb2 = pl.pallas_call(
    _comb2_body,
    out_shape=jax.ShapeDtypeStruct((N, D), jnp.float32),
)


def _lin_body(t_ref, w_ref, b_ref, o_ref):
    o_ref[...] = (jnp.dot(t_ref[...], w_ref[...],
                          preferred_element_type=jnp.float32) + b_ref[...])


_lin = pl.pallas_call(
    _lin_body,
    out_shape=jax.ShapeDtypeStruct((D, 8), jnp.float32),
)


def _band(w):
    w = w.reshape(3).astype(jnp.float32)
    return (jnp.eye(D, k=1, dtype=jnp.float32) * w[0]
            + jnp.eye(D, dtype=jnp.float32) * w[1]
            + jnp.eye(D, k=-1, dtype=jnp.float32) * w[2])


def kernel(x, edge_index, conv_w0, conv_b0, conv_w1, conv_b1, lin_w, lin_b):
    ei = edge_index.astype(jnp.int32)
    src = ei[0].reshape(NW, NSUP, S, K)
    dst = ei[1].reshape(NW, NSUP, S, K)
    dstf = ei[1].reshape(NW, NHS, 1, HS)
    c0 = _band(conv_w0)
    c1 = _band(conv_w1)
    b0 = jnp.full((1, D), conv_b0[0], jnp.float32)
    b1 = jnp.full((1, D), conv_b1[0], jnp.float32)

    h0 = _conv(x, c0, b0)
    p, cnt = _seg_counts(h0, src, dst, dstf)
    cnt = cnt.reshape(NW, N)
    h1 = _comb1(p, cnt, h0, c1, b1)
    p2 = _seg_plain(h1, src, dst)
    t = _comb2(p2, cnt, h1)

    r = t.reshape(D, N)
    wpad = jnp.concatenate(
        [lin_w.T.astype(jnp.float32), jnp.zeros((N, 5), jnp.float32)], axis=1)
    bpad = jnp.concatenate(
        [lin_b.astype(jnp.float32), jnp.zeros((5,), jnp.float32)])[None, :]
    out8 = _lin(r, wpad, bpad)
    return out8[:, :3]


# speedup vs baseline: 17.5335x; 1.0845x over previous
import jax
import jax.numpy as jnp
from jax import lax
from jax.experimental import pallas as pl
from jax.experimental.pallas import tpu as pltpu
from jax.experimental.pallas import tpu_sc as plsc

N = 10000
D = 128
E = 320000
NC = 2
NS = 16
NW = NC * NS
EPW = E // NW
K = 40
NCHUNK = EPW // K
S = 10
NSUP = NCHUNK // S
NB = 5
NGRP = S // NB
SLAB_STEP = 624
SLAB = 640


def _make_seg(with_counts):
    def body(*refs):
        if with_counts:
            (h_hbm, eir_hbm, p_hbm, cnt_hbm, accum,
             msg0, msg1, msg2, msg3, msg4, sidx0, sidx1, didx0, didx1,
             hist, gsem, ssem, isem) = refs
        else:
            (h_hbm, eir_hbm, p_hbm, accum,
             msg0, msg1, msg2, msg3, msg4, sidx0, sidx1, didx0, didx1,
             gsem, ssem, isem) = refs
        c = lax.axis_index("c")
        s = lax.axis_index("s")
        wid = s * NC + c
        row0 = s * SLAB_STEP
        msg = (msg0, msg1, msg2, msg3, msg4)
        sidx = (sidx0, sidx1)
        didx = (didx0, didx1)

        def pref_idx(sup):
            b = sup % 2
            pltpu.async_copy(eir_hbm.at[0, wid, sup], sidx[b], isem)
            pltpu.async_copy(eir_hbm.at[1, wid, sup], didx[b], isem)

        def wait_idx(sup):
            b = sup % 2
            pltpu.make_async_copy(eir_hbm.at[0, wid, sup], sidx[b],
                                  isem).wait()
            pltpu.make_async_copy(eir_hbm.at[1, wid, sup], didx[b],
                                  isem).wait()

        def gather(sb, g, b):
            return pltpu.async_copy(h_hbm.at[sb.at[g]], msg[b], gsem)

        def scat(db, g, b):
            pltpu.async_copy(msg[b], accum.at[db.at[g]], ssem, add=True)

        def wait_scat():
            pltpu.make_async_copy(msg[0], accum.at[didx0.at[0]], ssem).wait()

        if with_counts:
            ones16 = jnp.ones((16,), jnp.float32)
            zero16 = jnp.zeros((16,), jnp.int32)
            himask = jnp.arange(16, dtype=jnp.int32) >= 8

            def hist_rows(db):
                for r in range(S):
                    plsc.addupdate_scatter(
                        hist, [zero16, db[r, pl.ds(0, 16)]], ones16)
                    plsc.addupdate_scatter(
                        hist, [zero16, db[r, pl.ds(16, 16)]], ones16)
                    plsc.addupdate_scatter(
                        hist, [zero16, db[r, pl.ds(24, 16)]], ones16,
                        mask=himask)

        pref_idx(0)
        pltpu.sync_copy(h_hbm.at[pl.ds(row0, SLAB)],
                        accum.at[pl.ds(row0, SLAB)])
        if with_counts:
            def zero(i, _):
                hist[0, pl.ds(i * 16, 16)] = jnp.zeros((16,), jnp.float32)
                return 0
            lax.fori_loop(0, N // 16, zero, 0)
        plsc.subcore_barrier()

        for sup in range(NSUP):
            sb, db = sidx[sup % 2], didx[sup % 2]
            wait_idx(sup)
            cps = []
            for b in range(NB):
                if sup > 0:
                    wait_scat()
                cps.append(gather(sb, b, b))
            for b in range(NB):
                cps[b].wait()
                scat(db, b, b)
            if sup + 1 < NSUP:
                pref_idx(sup + 1)

            def grp(i, _, sb=sb, db=db):
                g0 = NB * i
                cps = []
                for b in range(NB):
                    wait_scat()
                    cps.append(gather(sb, g0 + b, b))
                for b in range(NB):
                    cps[b].wait()
                    scat(db, g0 + b, b)
                return 0

            lax.fori_loop(1, NGRP, grp, 0)
            if with_counts:
                hist_rows(db)

        for b in range(NB):
            wait_scat()
        if with_counts:
            pltpu.sync_copy(hist, cnt_hbm.at[wid])
        plsc.subcore_barrier()
        pltpu.sync_copy(accum.at[pl.ds(row0, SLAB)],
                        p_hbm.at[c].at[pl.ds(row0, SLAB)])

    out_type = [jax.ShapeDtypeStruct((NC, N, D), jnp.float32)]
    scratch = [
        pltpu.VMEM_SHARED((N, D), jnp.float32),
        pltpu.VMEM((K, D), jnp.float32),
        pltpu.VMEM((K, D), jnp.float32),
        pltpu.VMEM((K, D), jnp.float32),
        pltpu.VMEM((K, D), jnp.float32),
        pltpu.VMEM((K, D), jnp.float32),
        pltpu.VMEM((S, K), jnp.int32),
        pltpu.VMEM((S, K), jnp.int32),
        pltpu.VMEM((S, K), jnp.int32),
        pltpu.VMEM((S, K), jnp.int32),
    ]
    if with_counts:
        out_type = out_type + [jax.ShapeDtypeStruct((NW, 1, N), jnp.float32)]
        scratch = scratch + [pltpu.VMEM((1, N), jnp.float32)]
    scratch = scratch + [
        pltpu.SemaphoreType.DMA,
        pltpu.SemaphoreType.DMA,
        pltpu.SemaphoreType.DMA,
    ]
    return pl.kernel(
        body,
        out_type=tuple(out_type) if with_counts else out_type[0],
        mesh=plsc.VectorSubcoreMesh(core_axis_name="c", subcore_axis_name="s"),
        compiler_params=pltpu.CompilerParams(needs_layout_passes=False),
        scratch_types=scratch,
    )


_seg_counts = _make_seg(True)
_seg_plain = _make_seg(False)


def _cnt_recip(cnt_ref):
    ones = jnp.ones((NW, 8), jnp.float32)
    cnt = lax.dot_general(cnt_ref[...], ones, (((0,), (0,)), ((), ())),
                          preferred_element_type=jnp.float32)
    return 1.0 / (cnt[:, 0:1] + 1.0)


def _comb_body(p_ref, cnt_ref, h_ref, c_ref, b_ref, o_ref):
    rec = _cnt_recip(cnt_ref)
    m = (p_ref[0] + p_ref[1] - h_ref[...]) * rec
    o_ref[...] = jnp.maximum(
        jnp.dot(m, c_ref[...], preferred_element_type=jnp.float32)
        + b_ref[...], 0.0)


_comb = pl.pallas_call(
    _comb_body,
    out_shape=jax.ShapeDtypeStruct((N, D), jnp.float32),
)


def _lin_body(t_ref, w_ref, b_ref, o_ref):
    o_ref[...] = (jnp.dot(t_ref[...], w_ref[...],
                          preferred_element_type=jnp.float32) + b_ref[...])


_lin = pl.pallas_call(
    _lin_body,
    out_shape=jax.ShapeDtypeStruct((D, 8), jnp.float32),
)


def _band(w):
    w = w.reshape(3).astype(jnp.float32)
    return (jnp.eye(D, k=1, dtype=jnp.float32) * w[0]
            + jnp.eye(D, dtype=jnp.float32) * w[1]
            + jnp.eye(D, k=-1, dtype=jnp.float32) * w[2])


def kernel(x, edge_index, conv_w0, conv_b0, conv_w1, conv_b1, lin_w, lin_b):
    ei = edge_index.astype(jnp.int32)
    eir = ei.reshape(2, NW, NSUP, S, K)
    c0 = _band(conv_w0)
    c1 = _band(conv_w1)
    b0 = jnp.full((1, D), conv_b0[0], jnp.float32)
    b1 = jnp.full((1, D), conv_b1[0], jnp.float32)

    p, cnt = _seg_counts(x, eir)
    cnt = cnt.reshape(NW, N)
    h1 = _comb(p, cnt, x, c0, b0)
    p2 = _seg_plain(h1, eir)
    t = _comb(p2, cnt, h1, c1, b1)

    r = t.reshape(D, N)
    wpad = jnp.concatenate(
        [lin_w.T.astype(jnp.float32), jnp.zeros((N, 5), jnp.float32)], axis=1)
    bpad = jnp.concatenate(
        [lin_b.astype(jnp.float32), jnp.zeros((5,), jnp.float32)])[None, :]
    out8 = _lin(r, wpad, bpad)
    return out8[:, :3]


# speedup vs baseline: 17.6014x; 1.0039x over previous
import jax
import jax.numpy as jnp
from jax import lax
from jax.experimental import pallas as pl
from jax.experimental.pallas import tpu as pltpu
from jax.experimental.pallas import tpu_sc as plsc

N = 10000
D = 128
E = 320000
NC = 2
NS = 16
NW = NC * NS
EPW = E // NW
K = 50
NCHUNK = EPW // K
S = 8
NSUP = NCHUNK // S
NB = 4
NGRP = S // NB
SLAB_STEP = 624
SLAB = 640


def _make_seg(with_counts):
    def body(*refs):
        if with_counts:
            (h_hbm, eir_hbm, p_hbm, cnt_hbm, accum,
             msg0, msg1, msg2, msg3, sidx0, sidx1, didx0, didx1,
             hist, gsem, ssem, isem) = refs
        else:
            (h_hbm, eir_hbm, p_hbm, accum,
             msg0, msg1, msg2, msg3, sidx0, sidx1, didx0, didx1,
             gsem, ssem, isem) = refs
        c = lax.axis_index("c")
        s = lax.axis_index("s")
        wid = s * NC + c
        row0 = s * SLAB_STEP
        msg = (msg0, msg1, msg2, msg3)
        sidx = (sidx0, sidx1)
        didx = (didx0, didx1)

        def pref_idx(sup):
            b = sup % 2
            pltpu.async_copy(eir_hbm.at[0, wid, sup], sidx[b], isem)
            pltpu.async_copy(eir_hbm.at[1, wid, sup], didx[b], isem)

        def wait_idx(sup):
            b = sup % 2
            pltpu.make_async_copy(eir_hbm.at[0, wid, sup], sidx[b],
                                  isem).wait()
            pltpu.make_async_copy(eir_hbm.at[1, wid, sup], didx[b],
                                  isem).wait()

        def gather(sb, g, b):
            return pltpu.async_copy(h_hbm.at[sb.at[g]], msg[b], gsem)

        def scat(db, g, b):
            pltpu.async_copy(msg[b], accum.at[db.at[g]], ssem, add=True)

        def wait_scat():
            pltpu.make_async_copy(msg[0], accum.at[didx0.at[0]], ssem).wait()

        if with_counts:
            ones16 = jnp.ones((16,), jnp.float32)
            zero16 = jnp.zeros((16,), jnp.int32)
            himask = jnp.arange(16, dtype=jnp.int32) >= 14

            def hist_rows(db):
                for r in range(S):
                    for off in (0, 16, 32):
                        plsc.addupdate_scatter(
                            hist, [zero16, db[r, pl.ds(off, 16)]], ones16)
                    plsc.addupdate_scatter(
                        hist, [zero16, db[r, pl.ds(34, 16)]], ones16,
                        mask=himask)

        pref_idx(0)
        pltpu.sync_copy(h_hbm.at[pl.ds(row0, SLAB)],
                        accum.at[pl.ds(row0, SLAB)])
        if with_counts:
            def zero(i, _):
                hist[0, pl.ds(i * 16, 16)] = jnp.zeros((16,), jnp.float32)
                return 0
            lax.fori_loop(0, N // 16, zero, 0)
        plsc.subcore_barrier()

        for sup in range(NSUP):
            sb, db = sidx[sup % 2], didx[sup % 2]
            wait_idx(sup)
            cps = []
            for b in range(NB):
                if sup > 0:
                    wait_scat()
                cps.append(gather(sb, b, b))
            for b in range(NB):
                cps[b].wait()
                scat(db, b, b)
            if sup + 1 < NSUP:
                pref_idx(sup + 1)

            def grp(i, _, sb=sb, db=db):
                g0 = NB * i
                cps = []
                for b in range(NB):
                    wait_scat()
                    cps.append(gather(sb, g0 + b, b))
                for b in range(NB):
                    cps[b].wait()
                    scat(db, g0 + b, b)
                return 0

            lax.fori_loop(1, NGRP, grp, 0)
            if with_counts:
                hist_rows(db)

        for b in range(NB):
            wait_scat()
        if with_counts:
            pltpu.sync_copy(hist, cnt_hbm.at[wid])
        plsc.subcore_barrier()
        pltpu.sync_copy(accum.at[pl.ds(row0, SLAB)],
                        p_hbm.at[c].at[pl.ds(row0, SLAB)])

    out_type = [jax.ShapeDtypeStruct((NC, N, D), jnp.float32)]
    scratch = [
        pltpu.VMEM_SHARED((N, D), jnp.float32),
        pltpu.VMEM((K, D), jnp.float32),
        pltpu.VMEM((K, D), jnp.float32),
        pltpu.VMEM((K, D), jnp.float32),
        pltpu.VMEM((K, D), jnp.float32),
        pltpu.VMEM((S, K), jnp.int32),
        pltpu.VMEM((S, K), jnp.int32),
        pltpu.VMEM((S, K), jnp.int32),
        pltpu.VMEM((S, K), jnp.int32),
    ]
    if with_counts:
        out_type = out_type + [jax.ShapeDtypeStruct((NW, 1, N), jnp.float32)]
        scratch = scratch + [pltpu.VMEM((1, N), jnp.float32)]
    scratch = scratch + [
        pltpu.SemaphoreType.DMA,
        pltpu.SemaphoreType.DMA,
        pltpu.SemaphoreType.DMA,
    ]
    return pl.kernel(
        body,
        out_type=tuple(out_type) if with_counts else out_type[0],
        mesh=plsc.VectorSubcoreMesh(core_axis_name="c", subcore_axis_name="s"),
        compiler_params=pltpu.CompilerParams(needs_layout_passes=False),
        scratch_types=scratch,
    )


_seg_counts = _make_seg(True)
_seg_plain = _make_seg(False)


def _cnt_recip(cnt_ref):
    ones = jnp.ones((NW, 8), jnp.float32)
    cnt = lax.dot_general(cnt_ref[...], ones, (((0,), (0,)), ((), ())),
                          preferred_element_type=jnp.float32)
    return 1.0 / (cnt[:, 0:1] + 1.0)


def _comb_body(p_ref, cnt_ref, h_ref, c_ref, b_ref, o_ref):
    rec = _cnt_recip(cnt_ref)
    m = (p_ref[0] + p_ref[1] - h_ref[...]) * rec
    o_ref[...] = jnp.maximum(
        jnp.dot(m, c_ref[...], preferred_element_type=jnp.float32)
        + b_ref[...], 0.0)


_comb = pl.pallas_call(
    _comb_body,
    out_shape=jax.ShapeDtypeStruct((N, D), jnp.float32),
)


def _lin_body(t_ref, w_ref, b_ref, o_ref):
    o_ref[...] = (jnp.dot(t_ref[...], w_ref[...],
                          preferred_element_type=jnp.float32) + b_ref[...])


_lin = pl.pallas_call(
    _lin_body,
    out_shape=jax.ShapeDtypeStruct((D, 8), jnp.float32),
)


def _band(w):
    w = w.reshape(3).astype(jnp.float32)
    return (jnp.eye(D, k=1, dtype=jnp.float32) * w[0]
            + jnp.eye(D, dtype=jnp.float32) * w[1]
            + jnp.eye(D, k=-1, dtype=jnp.float32) * w[2])


def kernel(x, edge_index, conv_w0, conv_b0, conv_w1, conv_b1, lin_w, lin_b):
    ei = edge_index.astype(jnp.int32)
    eir = ei.reshape(2, NW, NSUP, S, K)
    c0 = _band(conv_w0)
    c1 = _band(conv_w1)
    b0 = jnp.full((1, D), conv_b0[0], jnp.float32)
    b1 = jnp.full((1, D), conv_b1[0], jnp.float32)

    p, cnt = _seg_counts(x, eir)
    cnt = cnt.reshape(NW, N)
    h1 = _comb(p, cnt, x, c0, b0)
    p2 = _seg_plain(h1, eir)
    t = _comb(p2, cnt, h1, c1, b1)

    r = t.reshape(D, N)
    wpad = jnp.concatenate(
        [lin_w.T.astype(jnp.float32), jnp.zeros((N, 5), jnp.float32)], axis=1)
    bpad = jnp.concatenate(
        [lin_b.astype(jnp.float32), jnp.zeros((5,), jnp.float32)])[None, :]
    out8 = _lin(r, wpad, bpad)
    return out8[:, :3]


# speedup vs baseline: 17.6938x; 1.0053x over previous
import jax
import jax.numpy as jnp
from jax import lax
from jax.experimental import pallas as pl
from jax.experimental.pallas import tpu as pltpu
from jax.experimental.pallas import tpu_sc as plsc

N = 10000
D = 128
E = 320000
NC = 2
NS = 16
NW = NC * NS
EPW = E // NW
K = 50
NCHUNK = EPW // K
S = 8
NSUP = NCHUNK // S
NB = 4
NGRP = S // NB
SLAB_STEP = 624
SLAB = 640


def _make_seg(with_counts):
    def body(*refs):
        if with_counts:
            (h_hbm, eir_hbm, p_hbm, cnt_hbm, accum,
             msg0, msg1, msg2, msg3, sidx0, sidx1, didx0, didx1,
             hist, gsem, ssem, isem) = refs
        else:
            (h_hbm, eir_hbm, p_hbm, accum,
             msg0, msg1, msg2, msg3, sidx0, sidx1, didx0, didx1,
             gsem, ssem, isem) = refs
        c = lax.axis_index("c")
        s = lax.axis_index("s")
        wid = s * NC + c
        row0 = s * SLAB_STEP
        msg = (msg0, msg1, msg2, msg3)
        sidx = (sidx0, sidx1)
        didx = (didx0, didx1)

        def pref_idx(sup):
            b = sup % 2
            pltpu.async_copy(eir_hbm.at[0, wid, sup], sidx[b], isem)
            pltpu.async_copy(eir_hbm.at[1, wid, sup], didx[b], isem)

        def wait_idx(sup):
            b = sup % 2
            pltpu.make_async_copy(eir_hbm.at[0, wid, sup], sidx[b],
                                  isem).wait()
            pltpu.make_async_copy(eir_hbm.at[1, wid, sup], didx[b],
                                  isem).wait()

        def gather(sb, g, b):
            return pltpu.async_copy(h_hbm.at[sb.at[g]], msg[b], gsem)

        def scat(db, g, b):
            pltpu.async_copy(msg[b], accum.at[db.at[g]], ssem, add=True)

        def wait_scat():
            pltpu.make_async_copy(msg[0], accum.at[didx0.at[0]], ssem).wait()

        if with_counts:
            ones16 = jnp.ones((16,), jnp.float32)
            zero16 = jnp.zeros((16,), jnp.int32)
            himask = jnp.arange(16, dtype=jnp.int32) >= 14

            def hist_rows(db):
                for r in range(S):
                    for off in (0, 16, 32):
                        plsc.addupdate_scatter(
                            hist, [zero16, db[r, pl.ds(off, 16)]], ones16)
                    plsc.addupdate_scatter(
                        hist, [zero16, db[r, pl.ds(34, 16)]], ones16,
                        mask=himask)

        pref_idx(0)
        pltpu.sync_copy(h_hbm.at[pl.ds(row0, SLAB)],
                        accum.at[pl.ds(row0, SLAB)])
        if with_counts:
            def zero(i, _):
                hist[0, pl.ds(i * 16, 16)] = jnp.zeros((16,), jnp.float32)
                return 0
            lax.fori_loop(0, N // 16, zero, 0)
        plsc.subcore_barrier()

        for sup in range(NSUP):
            sb, db = sidx[sup % 2], didx[sup % 2]
            wait_idx(sup)
            cps = []
            for b in range(NB):
                if sup > 0:
                    wait_scat()
                cps.append(gather(sb, b, b))
            for b in range(NB):
                cps[b].wait()
                scat(db, b, b)
            if sup + 1 < NSUP:
                pref_idx(sup + 1)
            if with_counts:
                hist_rows(db)

            def grp(i, _, sb=sb, db=db):
                g0 = NB * i
                cps = []
                for b in range(NB):
                    wait_scat()
                    cps.append(gather(sb, g0 + b, b))
                for b in range(NB):
                    cps[b].wait()
                    scat(db, g0 + b, b)
                return 0

            lax.fori_loop(1, NGRP, grp, 0)

        for b in range(NB):
            wait_scat()
        if with_counts:
            pltpu.sync_copy(hist, cnt_hbm.at[wid])
        plsc.subcore_barrier()
        pltpu.sync_copy(accum.at[pl.ds(row0, SLAB)],
                        p_hbm.at[c].at[pl.ds(row0, SLAB)])

    out_type = [jax.ShapeDtypeStruct((NC, N, D), jnp.float32)]
    scratch = [
        pltpu.VMEM_SHARED((N, D), jnp.float32),
        pltpu.VMEM((K, D), jnp.float32),
        pltpu.VMEM((K, D), jnp.float32),
        pltpu.VMEM((K, D), jnp.float32),
        pltpu.VMEM((K, D), jnp.float32),
        pltpu.VMEM((S, K), jnp.int32),
        pltpu.VMEM((S, K), jnp.int32),
        pltpu.VMEM((S, K), jnp.int32),
        pltpu.VMEM((S, K), jnp.int32),
    ]
    if with_counts:
        out_type = out_type + [jax.ShapeDtypeStruct((NW, 1, N), jnp.float32)]
        scratch = scratch + [pltpu.VMEM((1, N), jnp.float32)]
    scratch = scratch + [
        pltpu.SemaphoreType.DMA,
        pltpu.SemaphoreType.DMA,
        pltpu.SemaphoreType.DMA,
    ]
    return pl.kernel(
        body,
        out_type=tuple(out_type) if with_counts else out_type[0],
        mesh=plsc.VectorSubcoreMesh(core_axis_name="c", subcore_axis_name="s"),
        compiler_params=pltpu.CompilerParams(needs_layout_passes=False),
        scratch_types=scratch,
    )


_seg_counts = _make_seg(True)
_seg_plain = _make_seg(False)


def _cnt_recip(cnt_ref):
    ones = jnp.ones((NW, 8), jnp.float32)
    cnt = lax.dot_general(cnt_ref[...], ones, (((0,), (0,)), ((), ())),
                          preferred_element_type=jnp.float32)
    return 1.0 / (cnt[:, 0:1] + 1.0)


def _comb_body(p_ref, cnt_ref, h_ref, c_ref, b_ref, o_ref):
    rec = _cnt_recip(cnt_ref)
    m = (p_ref[0] + p_ref[1] - h_ref[...]) * rec
    o_ref[...] = jnp.maximum(
        jnp.dot(m, c_ref[...], preferred_element_type=jnp.float32)
        + b_ref[...], 0.0)


_comb = pl.pallas_call(
    _comb_body,
    out_shape=jax.ShapeDtypeStruct((N, D), jnp.float32),
)


def _lin_body(t_ref, w_ref, b_ref, o_ref):
    o_ref[...] = (jnp.dot(t_ref[...], w_ref[...],
                          preferred_element_type=jnp.float32) + b_ref[...])


_lin = pl.pallas_call(
    _lin_body,
    out_shape=jax.ShapeDtypeStruct((D, 8), jnp.float32),
)


def _band(w):
    w = w.reshape(3).astype(jnp.float32)
    return (jnp.eye(D, k=1, dtype=jnp.float32) * w[0]
            + jnp.eye(D, dtype=jnp.float32) * w[1]
            + jnp.eye(D, k=-1, dtype=jnp.float32) * w[2])


def kernel(x, edge_index, conv_w0, conv_b0, conv_w1, conv_b1, lin_w, lin_b):
    ei = edge_index.astype(jnp.int32)
    eir = ei.reshape(2, NW, NSUP, S, K)
    c0 = _band(conv_w0)
    c1 = _band(conv_w1)
    b0 = jnp.full((1, D), conv_b0[0], jnp.float32)
    b1 = jnp.full((1, D), conv_b1[0], jnp.float32)

    p, cnt = _seg_counts(x, eir)
    cnt = cnt.reshape(NW, N)
    h1 = _comb(p, cnt, x, c0, b0)
    p2 = _seg_plain(h1, eir)
    t = _comb(p2, cnt, h1, c1, b1)

    r = t.reshape(D, N)
    wpad = jnp.concatenate(
        [lin_w.T.astype(jnp.float32), jnp.zeros((N, 5), jnp.float32)], axis=1)
    bpad = jnp.concatenate(
        [lin_b.astype(jnp.float32), jnp.zeros((5,), jnp.float32)])[None, :]
    out8 = _lin(r, wpad, bpad)
    return out8[:, :3]
